# baseline (speedup 1.0000x reference)
import functools

import jax
import jax.numpy as jnp
from jax import lax
from jax.experimental import pallas as pl
from jax.experimental.pallas import tpu as pltpu
from jax.experimental.pallas import tpu_sc as plsc

_B = 16
_P = 4096
_S2 = 256
_RSC = 8
_RTC = _B - _RSC
_WPR = 4
_PSPAN = _P // _WPR
_PCH = 128
_NCH = _PSPAN // _PCH
_NLANE = 16

_SC_PARAMS = pltpu.CompilerParams(needs_layout_passes=False)


def _rowsum_body(x_ref, o_ref):
    x = x_ref[...]
    o_ref[...] = jnp.sum(x[0], axis=0)


def _patch_counts(xt):
    return pl.pallas_call(
        _rowsum_body,
        grid=(_RTC,),
        in_specs=[pl.BlockSpec((1, _S2, _P), lambda b: (b + _RSC, 0, 0))],
        out_specs=pl.BlockSpec((_P,), lambda b: (b,)),
        out_shape=jax.ShapeDtypeStruct((_RTC * _P,), jnp.int32),
    )(xt)


def _zero_hist(hist_v):
    zeros = jnp.zeros((_NLANE,), jnp.int32)
    for i in range(_NLANE * _S2 // _NLANE):
        hist_v[pl.ds(i * _NLANE, _NLANE)] = zeros


def _scatter_counts(hist_v, lane_base, c):
    ones = jnp.ones((_NLANE,), jnp.int32)
    plsc.addupdate_scatter(hist_v, [lane_base + c], ones, mask=c < _S2)


def _combine_and_write(out_hbm, hist_v, loc_v, part_v, probs_v, shr,
                       sid, sub, out_row):
    for j in range(_S2 // _NLANE):
        a = hist_v[pl.ds(j * _NLANE, _NLANE)]
        for l in range(1, _NLANE):
            a = a + hist_v[pl.ds(l * _S2 + j * _NLANE, _NLANE)]
        loc_v[pl.ds(j * _NLANE, _NLANE)] = a

    pltpu.sync_copy(loc_v, shr.at[sid])
    plsc.subcore_barrier()

    @pl.when(sub == 0)
    def _():
        for q in range(1, _WPR):
            pltpu.sync_copy(shr.at[sid + q * _WPR],
                            part_v.at[pl.ds((q - 1) * _S2, _S2)])
        total = jnp.zeros((_NLANE,), jnp.int32)
        accs = []
        for j in range(_S2 // _NLANE):
            a = loc_v[pl.ds(j * _NLANE, _NLANE)]
            for q in range(1, _WPR):
                a = a + part_v[pl.ds((q - 1) * _S2 + j * _NLANE, _NLANE)]
            total = total + a
            accs.append(a)
        t = jnp.sum(total)
        t_vec = lax.broadcast(t, (_NLANE,)).astype(jnp.float32)
        for j in range(_S2 // _NLANE):
            probs_v[pl.ds(j * _NLANE, _NLANE)] = (
                accs[j].astype(jnp.float32) / t_vec)
        pltpu.sync_copy(probs_v, out_hbm.at[pl.ds(out_row * _S2, _S2)])


def _sc_raw_body(xt_hbm, out_hbm, buf0, buf1, hist_v, loc_v, part_v,
                 probs_v, shr, sem0, sem1):
    cid = lax.axis_index("c")
    sid = lax.axis_index("s")
    rloc = lax.rem(sid, _WPR)
    row = cid * _WPR + rloc
    sub = sid // _WPR
    p_base = sub * _PSPAN

    _zero_hist(hist_v)
    lanes = lax.broadcasted_iota(jnp.int32, (_NLANE,), 0)
    lane_base = lanes * _S2
    zeros = jnp.zeros((_NLANE,), jnp.int32)

    bufs = (buf0, buf1)
    sems = (sem0, sem1)
    handles = [None, None]
    handles[0] = pltpu.async_copy(
        xt_hbm.at[row, :, pl.ds(p_base, _PCH)], buf0, sem0)
    for ch in range(_NCH):
        if ch + 1 < _NCH:
            handles[(ch + 1) % 2] = pltpu.async_copy(
                xt_hbm.at[row, :, pl.ds(p_base + (ch + 1) * _PCH, _PCH)],
                bufs[(ch + 1) % 2], sems[(ch + 1) % 2])
        handles[ch % 2].wait()
        b = bufs[ch % 2]

        def kbody(k, carry):
            return tuple(
                carry[j] + b[k, pl.ds(j * _NLANE, _NLANE)]
                for j in range(_PCH // _NLANE)
            )

        acc = lax.fori_loop(0, _S2, kbody,
                            tuple([zeros] * (_PCH // _NLANE)), unroll=8)
        for j in range(_PCH // _NLANE):
            _scatter_counts(hist_v, lane_base, acc[j])

    _combine_and_write(out_hbm, hist_v, loc_v, part_v, probs_v, shr,
                       sid, sub, row)


def _sc_probs_raw(xt):
    mesh = plsc.VectorSubcoreMesh(core_axis_name="c", subcore_axis_name="s")
    kern = functools.partial(
        pl.kernel,
        mesh=mesh,
        compiler_params=_SC_PARAMS,
        out_type=jax.ShapeDtypeStruct((_RSC * _S2,), jnp.float32),
        scratch_types=[
            pltpu.VMEM((_S2, _PCH), jnp.int32),
            pltpu.VMEM((_S2, _PCH), jnp.int32),
            pltpu.VMEM((_NLANE * _S2,), jnp.int32),
            pltpu.VMEM((_S2,), jnp.int32),
            pltpu.VMEM(((_WPR - 1) * _S2,), jnp.int32),
            pltpu.VMEM((_S2,), jnp.float32),
            pltpu.VMEM_SHARED((16, _S2), jnp.int32),
            pltpu.SemaphoreType.DMA,
            pltpu.SemaphoreType.DMA,
        ],
    )(_sc_raw_body)
    return kern(xt)


def _sc_counts_body(counts_hbm, out_hbm, row_v, hist_v, loc_v, part_v,
                    probs_v, shr):
    cid = lax.axis_index("c")
    sid = lax.axis_index("s")
    rloc = lax.rem(sid, _WPR)
    crow = cid * _WPR + rloc
    sub = sid // _WPR

    pltpu.sync_copy(
        counts_hbm.at[pl.ds(crow * _P + sub * _PSPAN, _PSPAN)], row_v)

    _zero_hist(hist_v)
    lanes = lax.broadcasted_iota(jnp.int32, (_NLANE,), 0)
    lane_base = lanes * _S2
    for j in range(_PSPAN // _NLANE):
        _scatter_counts(hist_v, lane_base, row_v[pl.ds(j * _NLANE, _NLANE)])

    _combine_and_write(out_hbm, hist_v, loc_v, part_v, probs_v, shr,
                       sid, sub, crow)


def _sc_probs_from_counts(counts):
    mesh = plsc.VectorSubcoreMesh(core_axis_name="c", subcore_axis_name="s")
    kern = functools.partial(
        pl.kernel,
        mesh=mesh,
        compiler_params=_SC_PARAMS,
        out_type=jax.ShapeDtypeStruct((_RTC * _S2,), jnp.float32),
        scratch_types=[
            pltpu.VMEM((_PSPAN,), jnp.int32),
            pltpu.VMEM((_NLANE * _S2,), jnp.int32),
            pltpu.VMEM((_S2,), jnp.int32),
            pltpu.VMEM(((_WPR - 1) * _S2,), jnp.int32),
            pltpu.VMEM((_S2,), jnp.float32),
            pltpu.VMEM_SHARED((16, _S2), jnp.int32),
        ],
    )(_sc_counts_body)
    return kern(counts)


def kernel(inputs):
    x = inputs
    xt = x.transpose(0, 1, 2, 4, 5, 3).reshape(_B, _S2, _P)
    probs_a = _sc_probs_raw(xt)
    counts = _patch_counts(xt)
    probs_b = _sc_probs_from_counts(counts)
    probs = jnp.concatenate([probs_a, probs_b]).reshape(_B, _S2)
    return ((probs,),)


# speedup vs baseline: 1.3800x; 1.3800x over previous
import functools

import jax
import jax.numpy as jnp
from jax import lax
from jax.experimental import pallas as pl
from jax.experimental.pallas import tpu as pltpu
from jax.experimental.pallas import tpu_sc as plsc

_B = 16
_P = 4096
_S2 = 256
_ROWS = _B * _P
_BLK = 2048
_GRID = _ROWS // _BLK


_RB = 2


def _rowsum_body(x_ref, o_ref):
    x = x_ref[...]
    for r in range(_RB):
        o_ref[pl.ds(r * _P, _P)] = jnp.sum(x[r], axis=0)


def _patch_counts(xt):
    return pl.pallas_call(
        _rowsum_body,
        grid=(_B // _RB,),
        in_specs=[pl.BlockSpec((_RB, _S2, _P), lambda b: (b, 0, 0))],
        out_specs=pl.BlockSpec((_RB * _P,), lambda b: (b,)),
        out_shape=jax.ShapeDtypeStruct((_ROWS,), jnp.int32),
    )(xt)


_HALF = _P // 2


def _hist_body(counts_hbm, out_hbm, row_v, hist_v, loc_v, part_v, probs_v, shr):
    cid = lax.axis_index("c")
    sid = lax.axis_index("s")
    row = cid * 8 + lax.rem(sid, 8)
    half = sid // 8

    pltpu.sync_copy(
        counts_hbm.at[pl.ds(row * _P + half * _HALF, _HALF)], row_v
    )

    zeros = jnp.zeros((16,), jnp.int32)
    for i in range(_B * _S2 // 16):
        hist_v[pl.ds(i * 16, 16)] = zeros

    lanes = lax.broadcasted_iota(jnp.int32, (16,), 0)
    lane_base = lanes * _S2
    ones = jnp.ones((16,), jnp.int32)

    for j in range(_HALF // 16):
        c = row_v[pl.ds(j * 16, 16)]
        plsc.addupdate_scatter(hist_v, [lane_base + c], ones, mask=c < _S2)

    for j in range(_S2 // 16):
        acc = hist_v[pl.ds(j * 16, 16)]
        for l in range(1, 16):
            acc = acc + hist_v[pl.ds(l * _S2 + j * 16, 16)]
        loc_v[pl.ds(j * 16, 16)] = acc

    @pl.when(half == 1)
    def _():
        pltpu.sync_copy(loc_v, shr.at[sid - 8])

    plsc.subcore_barrier()

    @pl.when(half == 0)
    def _():
        pltpu.sync_copy(shr.at[sid], part_v)
        total = jnp.zeros((16,), jnp.int32)
        accs = []
        for j in range(_S2 // 16):
            acc = loc_v[pl.ds(j * 16, 16)] + part_v[pl.ds(j * 16, 16)]
            total = total + acc
            accs.append(acc)
        t = jnp.sum(total)
        t_vec = lax.broadcast(t, (16,)).astype(jnp.float32)
        for j in range(_S2 // 16):
            probs_v[pl.ds(j * 16, 16)] = accs[j].astype(jnp.float32) / t_vec
        pltpu.sync_copy(probs_v, out_hbm.at[pl.ds(row * _S2, _S2)])


def _histogram_probs(counts):
    mesh = plsc.VectorSubcoreMesh(core_axis_name="c", subcore_axis_name="s")
    kern = functools.partial(
        pl.kernel,
        mesh=mesh,
        compiler_params=pltpu.CompilerParams(needs_layout_passes=False),
        out_type=jax.ShapeDtypeStruct((_B * _S2,), jnp.float32),
        scratch_types=[
            pltpu.VMEM((_HALF,), jnp.int32),
            pltpu.VMEM((_B * _S2,), jnp.int32),
            pltpu.VMEM((_S2,), jnp.int32),
            pltpu.VMEM((_S2,), jnp.int32),
            pltpu.VMEM((_S2,), jnp.float32),
            pltpu.VMEM_SHARED((8, _S2), jnp.int32),
        ],
    )(_hist_body)
    return kern(counts)


def kernel(inputs):
    x = inputs
    xt = x.transpose(0, 1, 2, 4, 5, 3).reshape(_B, _S2, _P)
    counts = _patch_counts(xt)
    probs = _histogram_probs(counts).reshape(_B, _S2)
    return ((probs,),)


# speedup vs baseline: 1.4218x; 1.0303x over previous
import functools

import jax
import jax.numpy as jnp
from jax import lax
from jax.experimental import pallas as pl
from jax.experimental.pallas import tpu as pltpu
from jax.experimental.pallas import tpu_sc as plsc

_B = 16
_P = 4096
_S2 = 256
_ROWS = _B * _P
_BLK = 2048
_GRID = _ROWS // _BLK


_RB = 2


def _rowsum_body(x_ref, o_ref):
    x = x_ref[...]
    for r in range(_RB):
        o_ref[pl.ds(r * _P, _P)] = jnp.sum(x[r], axis=0)


def _patch_counts(xt):
    return pl.pallas_call(
        _rowsum_body,
        grid=(_B // _RB,),
        in_specs=[pl.BlockSpec((_RB, _S2, _P), lambda b: (b, 0, 0))],
        out_specs=pl.BlockSpec((_RB * _P,), lambda b: (b,)),
        out_shape=jax.ShapeDtypeStruct((_ROWS,), jnp.int32),
    )(xt)


_HALF = _P // 2


def _hist_body(counts_hbm, out_hbm, row_v, loc_v, part_v, probs_v, shr):
    cid = lax.axis_index("c")
    sid = lax.axis_index("s")
    row = cid * 8 + lax.rem(sid, 8)
    half = sid // 8

    pltpu.sync_copy(
        counts_hbm.at[pl.ds(row * _P + half * _HALF, _HALF)], row_v
    )

    zeros = jnp.zeros((16,), jnp.int32)
    for i in range(_S2 // 16):
        loc_v[pl.ds(i * 16, 16)] = zeros

    ones = jnp.ones((16,), jnp.int32)

    for j in range(_HALF // 16):
        c = row_v[pl.ds(j * 16, 16)]
        plsc.addupdate_scatter(loc_v, [c], ones, mask=c < _S2)

    @pl.when(half == 1)
    def _():
        pltpu.sync_copy(loc_v, shr.at[sid - 8])

    plsc.subcore_barrier()

    @pl.when(half == 0)
    def _():
        pltpu.sync_copy(shr.at[sid], part_v)
        total = jnp.zeros((16,), jnp.int32)
        accs = []
        for j in range(_S2 // 16):
            acc = loc_v[pl.ds(j * 16, 16)] + part_v[pl.ds(j * 16, 16)]
            total = total + acc
            accs.append(acc)
        t = jnp.sum(total)
        t_vec = lax.broadcast(t, (16,)).astype(jnp.float32)
        for j in range(_S2 // 16):
            probs_v[pl.ds(j * 16, 16)] = accs[j].astype(jnp.float32) / t_vec
        pltpu.sync_copy(probs_v, out_hbm.at[pl.ds(row * _S2, _S2)])


def _histogram_probs(counts):
    mesh = plsc.VectorSubcoreMesh(core_axis_name="c", subcore_axis_name="s")
    kern = functools.partial(
        pl.kernel,
        mesh=mesh,
        compiler_params=pltpu.CompilerParams(needs_layout_passes=False),
        out_type=jax.ShapeDtypeStruct((_B * _S2,), jnp.float32),
        scratch_types=[
            pltpu.VMEM((_HALF,), jnp.int32),
            pltpu.VMEM((_S2,), jnp.int32),
            pltpu.VMEM((_S2,), jnp.int32),
            pltpu.VMEM((_S2,), jnp.float32),
            pltpu.VMEM_SHARED((8, _S2), jnp.int32),
        ],
    )(_hist_body)
    return kern(counts)


def kernel(inputs):
    x = inputs
    xt = x.transpose(0, 1, 2, 4, 5, 3).reshape(_B, _S2, _P)
    counts = _patch_counts(xt)
    probs = _histogram_probs(counts).reshape(_B, _S2)
    return ((probs,),)
